# Initial kernel scaffold; baseline (speedup 1.0000x reference)
import jax
import jax.numpy as jnp
from jax import lax
from jax.experimental import pallas as pl
from jax.experimental.pallas import tpu as pltpu
from jax.experimental.pallas import tpu_sc as plsc

B = 16384
L = 16
NS = 16
CHUNK = B // NS
NV = CHUNK // L
PAGE_SIZE = 16


def _body(seq_hbm, ll_hbm, fp_hbm, out_hbm,
          sl_v, ll_v, fp_v, ex_v, out_v, row_v, all_v, tot_sh):
    sid = lax.axis_index("s")
    base = sid * CHUNK

    pltpu.sync_copy(seq_hbm.at[pl.ds(base, CHUNK)], sl_v)
    pltpu.sync_copy(ll_hbm.at[pl.ds(base, CHUNK)], ll_v)
    pltpu.sync_copy(fp_hbm, fp_v)

    carry = jnp.int32(0)
    for j in range(NV):
        s = sl_v[pl.ds(j * L, L)]
        flag = jnp.where((s & (PAGE_SIZE - 1)) == 1, jnp.int32(1), jnp.int32(0))
        inc = plsc.cumsum(flag)
        ex_v[pl.ds(j * L, L)] = inc - flag + carry
        carry = carry + jnp.sum(flag)

    row_v[...] = jnp.full((L,), carry, jnp.int32)
    pltpu.sync_copy(row_v, tot_sh.at[sid])
    plsc.subcore_barrier()
    pltpu.sync_copy(tot_sh, all_v)

    lanes = lax.iota(jnp.int32, L)
    zeros = jnp.zeros((L,), jnp.int32)
    totals = plsc.load_gather(all_v, [lanes, zeros])
    my_base = jnp.sum(jnp.where(lanes < sid, totals, 0))

    for j in range(NV):
        idx = ex_v[pl.ds(j * L, L)] + my_base
        page = plsc.load_gather(fp_v, [idx])
        s = sl_v[pl.ds(j * L, L)]
        flag = (s & (PAGE_SIZE - 1)) == 1
        ll = ll_v[pl.ds(j * L, L)]
        out_v[pl.ds(j * L, L)] = jnp.where(flag, page * PAGE_SIZE, ll + 1)
    pltpu.sync_copy(out_v, out_hbm.at[pl.ds(base, CHUNK)])


@jax.jit
def _run(seq_lens, last_loc, free_page):
    mesh = plsc.VectorSubcoreMesh(
        core_axis_name="c", subcore_axis_name="s", num_cores=1)
    f = pl.kernel(
        _body,
        out_type=jax.ShapeDtypeStruct((B,), jnp.int32),
        mesh=mesh,
        scratch_types=[
            pltpu.VMEM((CHUNK,), jnp.int32),
            pltpu.VMEM((CHUNK,), jnp.int32),
            pltpu.VMEM((B,), jnp.int32),
            pltpu.VMEM((CHUNK,), jnp.int32),
            pltpu.VMEM((CHUNK,), jnp.int32),
            pltpu.VMEM((L,), jnp.int32),
            pltpu.VMEM((NS, L), jnp.int32),
            pltpu.VMEM_SHARED((NS, L), jnp.int32),
        ],
    )
    return f(seq_lens, last_loc, free_page)


def kernel(seq_lens, last_loc, free_page):
    return _run(seq_lens, last_loc, free_page)


# speedup vs baseline: 1.7312x; 1.7312x over previous
import jax
import jax.numpy as jnp
from jax import lax
from jax.experimental import pallas as pl
from jax.experimental.pallas import tpu as pltpu
from jax.experimental.pallas import tpu_sc as plsc

B = 16384
L = 16
NS = 16
CHUNK = B // NS
NV = CHUNK // L
PAGE_SIZE = 16


def _body(seq_hbm, ll_hbm, fp_hbm, out_hbm,
          sl_v, ll_v, fp_v, ex_v, out_v, cnt_s):
    sid = lax.axis_index("s")
    base = sid * CHUNK

    cnt_s[0] = jnp.int32(0)

    pltpu.sync_copy(seq_hbm.at[pl.ds(base, CHUNK)], sl_v)
    pltpu.sync_copy(ll_hbm.at[pl.ds(base, CHUNK)], ll_v)
    pltpu.sync_copy(fp_hbm, fp_v)

    carry = jnp.int32(0)
    for j in range(NV):
        s = sl_v[pl.ds(j * L, L)]
        flag = jnp.where((s & (PAGE_SIZE - 1)) == 1, jnp.int32(1), jnp.int32(0))
        inc = plsc.cumsum(flag)
        ex_v[pl.ds(j * L, L)] = inc - flag + carry
        carry = carry + jnp.sum(flag)

    plsc.subcore_barrier()
    for r in range(NS):
        val = jnp.where(sid < r, carry, jnp.int32(0))
        plsc.fetch_and_add(cnt_s.at[0], val, subcore_id=jnp.int32(r))
    plsc.subcore_barrier()
    my_base = cnt_s[0]

    for j in range(NV):
        idx = ex_v[pl.ds(j * L, L)] + my_base
        page = plsc.load_gather(fp_v, [idx])
        s = sl_v[pl.ds(j * L, L)]
        flag = (s & (PAGE_SIZE - 1)) == 1
        ll = ll_v[pl.ds(j * L, L)]
        out_v[pl.ds(j * L, L)] = jnp.where(flag, page * PAGE_SIZE, ll + 1)
    pltpu.sync_copy(out_v, out_hbm.at[pl.ds(base, CHUNK)])


@jax.jit
def _run(seq_lens, last_loc, free_page):
    mesh = plsc.VectorSubcoreMesh(
        core_axis_name="c", subcore_axis_name="s", num_cores=1, num_subcores=NS)
    f = pl.kernel(
        _body,
        out_type=jax.ShapeDtypeStruct((B,), jnp.int32),
        mesh=mesh,
        compiler_params=pltpu.CompilerParams(needs_layout_passes=False),
        scratch_types=[
            pltpu.VMEM((CHUNK,), jnp.int32),
            pltpu.VMEM((CHUNK,), jnp.int32),
            pltpu.VMEM((B,), jnp.int32),
            pltpu.VMEM((CHUNK,), jnp.int32),
            pltpu.VMEM((CHUNK,), jnp.int32),
            pltpu.SMEM((1,), jnp.int32),
        ],
    )
    return f(seq_lens, last_loc, free_page)


def kernel(seq_lens, last_loc, free_page):
    return _run(seq_lens, last_loc, free_page)


# speedup vs baseline: 1.8331x; 1.0589x over previous
import jax
import jax.numpy as jnp
from jax import lax
from jax.experimental import pallas as pl
from jax.experimental.pallas import tpu as pltpu
from jax.experimental.pallas import tpu_sc as plsc

B = 16384
L = 16
NS = 16
CHUNK = B // NS
NV = CHUNK // L
PAGE_SIZE = 16


def _body(seq_hbm, ll_hbm, fp_hbm, out_hbm,
          sl_v, ll_v, fp_v, ex_v, out_v, cnt_s, sem_fp, sem_ll):
    sid = lax.axis_index("s")
    base = sid * CHUNK

    cnt_s[0] = jnp.int32(0)

    fp_cp = pltpu.async_copy(fp_hbm, fp_v, sem_fp)
    ll_cp = pltpu.async_copy(ll_hbm.at[pl.ds(base, CHUNK)], ll_v, sem_ll)
    pltpu.sync_copy(seq_hbm.at[pl.ds(base, CHUNK)], sl_v)

    carry = jnp.int32(0)
    for j in range(NV):
        s = sl_v[pl.ds(j * L, L)]
        flag = jnp.where((s & (PAGE_SIZE - 1)) == 1, jnp.int32(1), jnp.int32(0))
        inc = plsc.cumsum(flag)
        ex_v[pl.ds(j * L, L)] = inc - flag + carry
        carry = carry + jnp.sum(flag)

    plsc.subcore_barrier()
    for r in range(1, NS):
        val = jnp.where(sid < r, carry, jnp.int32(0))
        plsc.fetch_and_add(cnt_s.at[0], val, subcore_id=jnp.int32(r))
    plsc.subcore_barrier()
    my_base = cnt_s[0]

    fp_cp.wait()
    ll_cp.wait()

    for j in range(NV):
        idx = ex_v[pl.ds(j * L, L)] + my_base
        page = plsc.load_gather(fp_v, [idx])
        s = sl_v[pl.ds(j * L, L)]
        flag = (s & (PAGE_SIZE - 1)) == 1
        ll = ll_v[pl.ds(j * L, L)]
        out_v[pl.ds(j * L, L)] = jnp.where(flag, page * PAGE_SIZE, ll + 1)
    pltpu.sync_copy(out_v, out_hbm.at[pl.ds(base, CHUNK)])


@jax.jit
def _run(seq_lens, last_loc, free_page):
    mesh = plsc.VectorSubcoreMesh(
        core_axis_name="c", subcore_axis_name="s", num_cores=1, num_subcores=NS)
    f = pl.kernel(
        _body,
        out_type=jax.ShapeDtypeStruct((B,), jnp.int32),
        mesh=mesh,
        compiler_params=pltpu.CompilerParams(needs_layout_passes=False),
        scratch_types=[
            pltpu.VMEM((CHUNK,), jnp.int32),
            pltpu.VMEM((CHUNK,), jnp.int32),
            pltpu.VMEM((B,), jnp.int32),
            pltpu.VMEM((CHUNK,), jnp.int32),
            pltpu.VMEM((CHUNK,), jnp.int32),
            pltpu.SMEM((1,), jnp.int32),
            pltpu.SemaphoreType.DMA,
            pltpu.SemaphoreType.DMA,
        ],
    )
    return f(seq_lens, last_loc, free_page)


def kernel(seq_lens, last_loc, free_page):
    return _run(seq_lens, last_loc, free_page)


# speedup vs baseline: 1.8697x; 1.0199x over previous
import jax
import jax.numpy as jnp
from jax import lax
from jax.experimental import pallas as pl
from jax.experimental.pallas import tpu as pltpu
from jax.experimental.pallas import tpu_sc as plsc

B = 16384
L = 16
NS = 16
CHUNK = B // NS
NV = CHUNK // L
PAGE_SIZE = 16
WIN = CHUNK + 8


def _body(seq_hbm, ll_hbm, fp_hbm, out_hbm,
          sl_v, ll_v, fp_v, ex_v, out_v, cnt_s, sem_fp, sem_ll):
    sid = lax.axis_index("s")
    base = sid * CHUNK

    cnt_s[0] = jnp.int32(0)

    ll_cp = pltpu.async_copy(ll_hbm.at[pl.ds(base, CHUNK)], ll_v, sem_ll)
    pltpu.sync_copy(seq_hbm.at[pl.ds(base, CHUNK)], sl_v)

    carry = jnp.int32(0)
    for j in range(NV):
        s = sl_v[pl.ds(j * L, L)]
        flag = jnp.where((s & (PAGE_SIZE - 1)) == 1, jnp.int32(1), jnp.int32(0))
        inc = plsc.cumsum(flag)
        ex_v[pl.ds(j * L, L)] = inc - flag + carry
        carry = carry + jnp.sum(flag)

    plsc.subcore_barrier()
    for r in range(1, NS):
        val = jnp.where(sid < r, carry, jnp.int32(0))
        plsc.fetch_and_add(cnt_s.at[0], val, subcore_id=jnp.int32(r))
    plsc.subcore_barrier()
    my_base = cnt_s[0]

    off = jnp.minimum(my_base, jnp.int32(B - WIN))
    off = pl.multiple_of((off // 8) * 8, 8)
    fp_cp = pltpu.async_copy(fp_hbm.at[pl.ds(off, WIN)], fp_v, sem_fp)
    rel = my_base - off

    ll_cp.wait()
    fp_cp.wait()

    for j in range(NV):
        idx = ex_v[pl.ds(j * L, L)] + rel
        page = plsc.load_gather(fp_v, [idx])
        s = sl_v[pl.ds(j * L, L)]
        flag = (s & (PAGE_SIZE - 1)) == 1
        ll = ll_v[pl.ds(j * L, L)]
        out_v[pl.ds(j * L, L)] = jnp.where(flag, page * PAGE_SIZE, ll + 1)
    pltpu.sync_copy(out_v, out_hbm.at[pl.ds(base, CHUNK)])


@jax.jit
def _run(seq_lens, last_loc, free_page):
    mesh = plsc.VectorSubcoreMesh(
        core_axis_name="c", subcore_axis_name="s", num_cores=1, num_subcores=NS)
    f = pl.kernel(
        _body,
        out_type=jax.ShapeDtypeStruct((B,), jnp.int32),
        mesh=mesh,
        compiler_params=pltpu.CompilerParams(needs_layout_passes=False),
        scratch_types=[
            pltpu.VMEM((CHUNK,), jnp.int32),
            pltpu.VMEM((CHUNK,), jnp.int32),
            pltpu.VMEM((WIN,), jnp.int32),
            pltpu.VMEM((CHUNK,), jnp.int32),
            pltpu.VMEM((CHUNK,), jnp.int32),
            pltpu.SMEM((1,), jnp.int32),
            pltpu.SemaphoreType.DMA,
            pltpu.SemaphoreType.DMA,
        ],
    )
    return f(seq_lens, last_loc, free_page)


def kernel(seq_lens, last_loc, free_page):
    return _run(seq_lens, last_loc, free_page)


# speedup vs baseline: 1.9081x; 1.0205x over previous
import jax
import jax.numpy as jnp
from jax import lax
from jax.experimental import pallas as pl
from jax.experimental.pallas import tpu as pltpu
from jax.experimental.pallas import tpu_sc as plsc

B = 16384
L = 16
NS = 16
CHUNK = B // NS
NV = CHUNK // L
PAGE_SIZE = 16
WIN = CHUNK + 8


def _body(seq_hbm, ll_hbm, fp_hbm, out_hbm,
          sl_v, ll_v, fp_v, ex_v, out_v, cnt_s, sem_fp, sem_ll):
    sid = lax.axis_index("s")
    base = sid * CHUNK

    for d_i in range(4):
        cnt_s[d_i] = jnp.int32(0)

    ll_cp = pltpu.async_copy(ll_hbm.at[pl.ds(base, CHUNK)], ll_v, sem_ll)
    pltpu.sync_copy(seq_hbm.at[pl.ds(base, CHUNK)], sl_v)

    carry = jnp.int32(0)
    for j in range(NV):
        s = sl_v[pl.ds(j * L, L)]
        flag = jnp.where((s & (PAGE_SIZE - 1)) == 1, jnp.int32(1), jnp.int32(0))
        inc = plsc.cumsum(flag)
        ex_v[pl.ds(j * L, L)] = inc - flag + carry
        carry = carry + jnp.sum(flag)

    v = carry
    plsc.subcore_barrier()
    for d_i, d in enumerate((1, 2, 4, 8)):
        val = jnp.where(sid + d < NS, v, jnp.int32(0))
        tgt = (sid + d) & (NS - 1)
        plsc.fetch_and_add(cnt_s.at[d_i], val, subcore_id=tgt)
        plsc.subcore_barrier()
        v = v + cnt_s[d_i]
    my_base = v - carry

    off = jnp.minimum(my_base, jnp.int32(B - WIN))
    off = pl.multiple_of((off // 8) * 8, 8)
    fp_cp = pltpu.async_copy(fp_hbm.at[pl.ds(off, WIN)], fp_v, sem_fp)
    rel = my_base - off

    ll_cp.wait()
    fp_cp.wait()

    for j in range(NV):
        idx = ex_v[pl.ds(j * L, L)] + rel
        page = plsc.load_gather(fp_v, [idx])
        s = sl_v[pl.ds(j * L, L)]
        flag = (s & (PAGE_SIZE - 1)) == 1
        ll = ll_v[pl.ds(j * L, L)]
        out_v[pl.ds(j * L, L)] = jnp.where(flag, page * PAGE_SIZE, ll + 1)
    pltpu.sync_copy(out_v, out_hbm.at[pl.ds(base, CHUNK)])


@jax.jit
def _run(seq_lens, last_loc, free_page):
    mesh = plsc.VectorSubcoreMesh(
        core_axis_name="c", subcore_axis_name="s", num_cores=1, num_subcores=NS)
    f = pl.kernel(
        _body,
        out_type=jax.ShapeDtypeStruct((B,), jnp.int32),
        mesh=mesh,
        compiler_params=pltpu.CompilerParams(needs_layout_passes=False),
        scratch_types=[
            pltpu.VMEM((CHUNK,), jnp.int32),
            pltpu.VMEM((CHUNK,), jnp.int32),
            pltpu.VMEM((WIN,), jnp.int32),
            pltpu.VMEM((CHUNK,), jnp.int32),
            pltpu.VMEM((CHUNK,), jnp.int32),
            pltpu.SMEM((4,), jnp.int32),
            pltpu.SemaphoreType.DMA,
            pltpu.SemaphoreType.DMA,
        ],
    )
    return f(seq_lens, last_loc, free_page)


def kernel(seq_lens, last_loc, free_page):
    return _run(seq_lens, last_loc, free_page)


# speedup vs baseline: 1.9271x; 1.0099x over previous
import jax
import jax.numpy as jnp
from jax import lax
from jax.experimental import pallas as pl
from jax.experimental.pallas import tpu as pltpu
from jax.experimental.pallas import tpu_sc as plsc

B = 16384
L = 16
NS = 16
CHUNK = B // NS
NV = CHUNK // L
PAGE_SIZE = 16
WIN = CHUNK + 8


def _body(seq_hbm, ll_hbm, fp_hbm, out_hbm,
          sl_v, ll_v, fp_v, fl_v, ex_v, out_v, cnt_s, sem_fp, sem_ll):
    sid = lax.axis_index("s")
    base = sid * CHUNK

    for d_i in range(4):
        cnt_s[d_i] = jnp.int32(0)

    ll_cp = pltpu.async_copy(ll_hbm.at[pl.ds(base, CHUNK)], ll_v, sem_ll)
    pltpu.sync_copy(seq_hbm.at[pl.ds(base, CHUNK)], sl_v)

    acc = jnp.zeros((L,), jnp.int32)
    for j in range(NV):
        s = sl_v[pl.ds(j * L, L)]
        flag = jnp.where((s & (PAGE_SIZE - 1)) == 1, jnp.int32(1), jnp.int32(0))
        fl_v[pl.ds(j * L, L)] = flag
        acc = acc + flag
    carry = jnp.sum(acc)

    v = carry
    plsc.subcore_barrier()
    for d_i, d in enumerate((1, 2, 4, 8)):
        val = jnp.where(sid + d < NS, v, jnp.int32(0))
        tgt = (sid + d) & (NS - 1)
        plsc.fetch_and_add(cnt_s.at[d_i], val, subcore_id=tgt)
        plsc.subcore_barrier()
        v = v + cnt_s[d_i]
    my_base = v - carry

    off = jnp.minimum(my_base, jnp.int32(B - WIN))
    off = pl.multiple_of((off // 8) * 8, 8)
    fp_cp = pltpu.async_copy(fp_hbm.at[pl.ds(off, WIN)], fp_v, sem_fp)
    rel = my_base - off

    run = rel
    for j in range(NV):
        f = fl_v[pl.ds(j * L, L)]
        inc = plsc.cumsum(f)
        ex_v[pl.ds(j * L, L)] = inc - f + run
        run = run + jnp.sum(f)

    ll_cp.wait()
    fp_cp.wait()

    for j in range(NV):
        f = fl_v[pl.ds(j * L, L)]
        idx = ex_v[pl.ds(j * L, L)]
        page = plsc.load_gather(fp_v, [idx])
        ll = ll_v[pl.ds(j * L, L)]
        out_v[pl.ds(j * L, L)] = jnp.where(f == 1, page * PAGE_SIZE, ll + 1)
    pltpu.sync_copy(out_v, out_hbm.at[pl.ds(base, CHUNK)])


@jax.jit
def _run(seq_lens, last_loc, free_page):
    mesh = plsc.VectorSubcoreMesh(
        core_axis_name="c", subcore_axis_name="s", num_cores=1, num_subcores=NS)
    f = pl.kernel(
        _body,
        out_type=jax.ShapeDtypeStruct((B,), jnp.int32),
        mesh=mesh,
        compiler_params=pltpu.CompilerParams(needs_layout_passes=False),
        scratch_types=[
            pltpu.VMEM((CHUNK,), jnp.int32),
            pltpu.VMEM((CHUNK,), jnp.int32),
            pltpu.VMEM((WIN,), jnp.int32),
            pltpu.VMEM((CHUNK,), jnp.int32),
            pltpu.VMEM((CHUNK,), jnp.int32),
            pltpu.VMEM((CHUNK,), jnp.int32),
            pltpu.SMEM((4,), jnp.int32),
            pltpu.SemaphoreType.DMA,
            pltpu.SemaphoreType.DMA,
        ],
    )
    return f(seq_lens, last_loc, free_page)


def kernel(seq_lens, last_loc, free_page):
    return _run(seq_lens, last_loc, free_page)
